# Initial kernel scaffold; baseline (speedup 1.0000x reference)
import jax
import jax.numpy as jnp
from jax.experimental import pallas as pl


def kernel(x, edge_index, edge_attr, params):
    raise NotImplementedError("write your pallas kernel here")


# speedup vs baseline: 1.0003x; 1.0003x over previous
import jax
import jax.numpy as jnp
from jax.experimental import pallas as pl

N_NODES = 10000
HID = 64


def _readout_body(r_ref, w1_ref, b1_ref, w2_ref, b2_ref, o_ref):
    o = jax.nn.relu(r_ref[...] @ w1_ref[...] + b1_ref[...])
    o_ref[...] = o @ w2_ref[...] + b2_ref[...]


def kernel(x, edge_index, edge_attr, params):
    h = jax.nn.relu(x @ params["in_W"] + params["in_b"])
    src = edge_index[0]
    dst = edge_index[1]
    ones = jnp.ones((edge_index.shape[1],), dtype=jnp.float32)
    deg_raw = jax.ops.segment_sum(ones, dst, num_segments=N_NODES)
    has_edge = deg_raw > 0
    deg = jnp.maximum(deg_raw, 1.0)[:, None]
    for layer in params["layers"]:
        h_in = h
        m = jnp.concatenate([h[src], h[dst], edge_attr], axis=-1)
        m = jax.nn.relu(m @ layer["pre_W"] + layer["pre_b"])
        s = jax.ops.segment_sum(m, dst, num_segments=N_NODES)
        mean_agg = s / deg
        max_agg = jax.ops.segment_max(m, dst, num_segments=N_NODES)
        max_agg = jnp.where(has_edge[:, None], max_agg, 0.0)
        agg = jnp.concatenate([mean_agg, max_agg], axis=-1)
        h = jnp.concatenate([h_in, agg], axis=-1) @ layer["post_W"] + layer["post_b"]
        h = h + h_in
    r = jnp.concatenate(
        [jnp.sum(h, axis=0), jnp.mean(h, axis=0), jnp.max(h, axis=0)], axis=-1
    )[None, :]
    out = pl.pallas_call(
        _readout_body,
        out_shape=jax.ShapeDtypeStruct((1, params["ro2_b"].shape[0]), jnp.float32),
    )(r, params["ro1_W"], params["ro1_b"][None, :], params["ro2_W"], params["ro2_b"][None, :])
    return out


# speedup vs baseline: 4.0843x; 4.0832x over previous
import dataclasses
import functools

import jax
import jax.numpy as jnp
from jax import lax
from jax.experimental import pallas as pl
from jax.experimental.pallas import tpu as pltpu
from jax.experimental.pallas import tpu_sc as plsc

N_NODES = 10000
NP = 10240
TPN = 320
E = 320000
E_SORT = 321536
E_PHYS = E_SORT + 512
C = 384
HID = 64
PADV = NP - 1


def _in_body(x_ref, w_ref, b_ref, h_ref):
    h_ref[...] = jax.nn.relu(
        jnp.dot(x_ref[...], w_ref[...], preferred_element_type=jnp.float32)
        + b_ref[...])


def _proj_body(h_ref, ws_ref, wd_ref, b_ref, hs_ref, hd_ref):
    h = h_ref[...]
    hs_ref[...] = jnp.dot(h, ws_ref[...], preferred_element_type=jnp.float32)
    hd_ref[...] = jnp.dot(h, wd_ref[...], preferred_element_type=jnp.float32) + b_ref[...]


def _eproj_body(ea_ref, w_ref, o0, o1, o2, o3, o4):
    ea = ea_ref[...]
    outs = (o0, o1, o2, o3, o4)
    for l in range(5):
        outs[l][...] = jnp.dot(ea, w_ref[l], preferred_element_type=jnp.float32)


def _post_body(h_ref, s_ref, m_ref, deg_ref, ph_ref, pm_ref, px_ref, b_ref, o_ref):
    h = h_ref[...]
    inv = 1.0 / jnp.maximum(deg_ref[...], 1.0)
    mean = s_ref[...] * inv
    o = (jnp.dot(h, ph_ref[...], preferred_element_type=jnp.float32)
         + jnp.dot(mean, pm_ref[...], preferred_element_type=jnp.float32)
         + jnp.dot(m_ref[...], px_ref[...], preferred_element_type=jnp.float32)
         + b_ref[...])
    o_ref[...] = o + h


def _readout_body(h_ref, w1_ref, b1_ref, w2_ref, b2_ref, o_ref):
    rows = lax.broadcasted_iota(jnp.int32, (NP, 1), 0)
    valid = rows < N_NODES
    h = h_ref[...]
    hs = jnp.where(valid, h, 0.0)
    hm = jnp.where(valid, h, -jnp.inf)
    s = jnp.sum(hs, axis=0, keepdims=True)
    mx = jnp.max(hm, axis=0, keepdims=True)
    r = jnp.concatenate([s, s * (1.0 / N_NODES), mx], axis=1)
    o = jax.nn.relu(
        jnp.dot(r, w1_ref[...], preferred_element_type=jnp.float32) + b1_ref[...])
    o_ref[...] = jnp.dot(o, w2_ref[...], preferred_element_type=jnp.float32) + b2_ref[...]


def _tc_in(x_pad, w, b):
    return pl.pallas_call(
        _in_body,
        out_shape=jax.ShapeDtypeStruct((NP, HID), jnp.float32),
    )(x_pad, w, b[None, :])


def _tc_proj(h, ws, wd, b):
    return pl.pallas_call(
        _proj_body,
        out_shape=[jax.ShapeDtypeStruct((NP, HID), jnp.float32)] * 2,
    )(h, ws, wd, b[None, :])


def _tc_eproj(ea_pad, w_stack):
    blk = 2048
    grid = E_SORT // blk
    return pl.pallas_call(
        _eproj_body,
        grid=(grid,),
        in_specs=[
            pl.BlockSpec((blk, 16), lambda i: (i, 0)),
            pl.BlockSpec((5, 16, HID), lambda i: (0, 0, 0)),
        ],
        out_specs=[pl.BlockSpec((blk, HID), lambda i: (i, 0))] * 5,
        out_shape=[jax.ShapeDtypeStruct((E_SORT, HID), jnp.float32)] * 5,
    )(ea_pad, w_stack)


def _tc_post(h, s, m, deg, ph, pm, px, b):
    return pl.pallas_call(
        _post_body,
        out_shape=jax.ShapeDtypeStruct((NP, HID), jnp.float32),
    )(h, s, m, deg, ph, pm, px, b[None, :])


def _tc_readout(h, w1, b1, w2, b2):
    return pl.pallas_call(
        _readout_body,
        out_shape=jax.ShapeDtypeStruct((1, w2.shape[1]), jnp.float32),
    )(h, w1, b1[None, :], w2, b2[None, :])


_MESH = plsc.VectorSubcoreMesh(core_axis_name="c", subcore_axis_name="s")
_Z16 = None


def _make_sc_edge(want_deg):
    out_type = [jax.ShapeDtypeStruct((NP, HID), jnp.float32)] * 2
    if want_deg:
        out_type = out_type + [jax.ShapeDtypeStruct((NP,), jnp.float32)]
    scratch = [
        pltpu.VMEM((48,), jnp.int32),
        pltpu.VMEM((C,), jnp.int32),
        pltpu.VMEM((C,), jnp.int32),
        pltpu.VMEM((C,), jnp.int32),
        pltpu.VMEM((C, HID), jnp.float32),
        pltpu.VMEM((C, HID), jnp.float32),
        pltpu.VMEM((TPN + 1, HID), jnp.float32),
        pltpu.VMEM((TPN + 1, HID), jnp.float32),
        pltpu.VMEM((TPN + 1, HID), jnp.float32),
        pltpu.VMEM((336,), jnp.float32),
    ] + [pltpu.SemaphoreType.DMA] * 8

    def body(dst_hbm, src_hbm, perm_hbm, hs_hbm, hd_hbm, ep_hbm, bounds_hbm,
             osum_hbm, omax_hbm, *rest):
        if want_deg:
            odeg_hbm = rest[0]
            rest = rest[1:]
        (bv, didx, sidx, pidx, hsv, epv, hdv, accs, accm, degv,
         s0, s1, s2, s3, s4, s5, s6, s7) = rest
        sems = (s0, s1, s2, s3, s4, s5, s6, s7)

        wid = lax.axis_index("s") * 2 + lax.axis_index("c")
        n0 = wid * TPN
        pltpu.sync_copy(bounds_hbm, bv)
        bwin = bv[pl.ds(wid, 16)]
        e0 = bwin[0]
        e1 = bwin[1]
        e0a = (e0 // 8) * 8
        nch = (e1 - e0a + (C - 1)) // C

        zero16 = jnp.zeros((16,), jnp.float32)

        @pl.loop(0, TPN + 1)
        def _(r):
            for j in range(4):
                accs[r, pl.ds(j * 16, 16)] = zero16
                accm[r, pl.ds(j * 16, 16)] = zero16

        if want_deg:
            @pl.loop(0, 336 // 16)
            def _(r):
                degv[pl.ds(r * 16, 16)] = zero16

        pltpu.sync_copy(hd_hbm.at[pl.ds(n0, TPN)], hdv.at[pl.ds(0, TPN)])
        for j in range(4):
            hdv[TPN, pl.ds(j * 16, 16)] = zero16

        def chunk(k, carry):
            ec = e0a + k * C
            cd = pltpu.async_copy(dst_hbm.at[pl.ds(ec, C)], didx, sems[0])
            cs = pltpu.async_copy(src_hbm.at[pl.ds(ec, C)], sidx, sems[1])
            cp = pltpu.async_copy(perm_hbm.at[pl.ds(ec, C)], pidx, sems[2])
            cd.wait()
            cs.wait()
            cp.wait()
            gs = []
            for s in range(C // 128):
                sl = pl.ds(s * 128, 128)
                gs.append(pltpu.async_copy(
                    hs_hbm.at[sidx.at[sl]], hsv.at[sl], sems[2 * s]))
                gs.append(pltpu.async_copy(
                    ep_hbm.at[pidx.at[sl]], epv.at[sl], sems[2 * s + 1]))
            for g in gs:
                g.wait()

            if want_deg:
                @pl.loop(0, C // 16)
                def _(i2):
                    dv = didx[pl.ds(i2 * 16, 16)]
                    dl = dv - n0
                    ok = (dl >= 0) & (dl < TPN)
                    idx = jnp.where(ok, dl, TPN)
                    ones = jnp.where(ok, 1.0, 0.0).astype(jnp.float32)
                    plsc.addupdate_scatter(degv, [idx], ones)

            @pl.loop(0, C // 16)
            def _(i2):
                dvec = didx[pl.ds(i2 * 16, 16)]
                dlv = dvec - n0
                okv = (dlv >= 0) & (dlv < TPN)
                rv = jnp.where(okv, dlv, TPN)
                for lane in range(16):
                    r = rv[lane]
                    ei = i2 * 16 + lane
                    for j in range(4):
                        sl = pl.ds(j * 16, 16)
                        v = hsv[ei, sl] + epv[ei, sl] + hdv[r, sl]
                        v = jnp.maximum(v, 0.0)
                        accs[r, sl] += v
                        accm[r, sl] = jnp.maximum(accm[r, sl], v)
            return carry

        lax.fori_loop(0, nch, chunk, 0)

        pltpu.sync_copy(accs.at[pl.ds(0, TPN)], osum_hbm.at[pl.ds(n0, TPN)])
        pltpu.sync_copy(accm.at[pl.ds(0, TPN)], omax_hbm.at[pl.ds(n0, TPN)])
        if want_deg:
            pltpu.sync_copy(degv.at[pl.ds(0, TPN)], odeg_hbm.at[pl.ds(n0, TPN)])

    cp = pltpu.CompilerParams(needs_layout_passes=False,
                              use_tc_tiling_on_sc=False)
    return pl.kernel(body, out_type=out_type, mesh=_MESH, scratch_types=scratch,
                     compiler_params=cp)


_sc_edge_deg = _make_sc_edge(True)
_sc_edge = _make_sc_edge(False)


def kernel(x, edge_index, edge_attr, params):
    src = edge_index[0].astype(jnp.int32)
    dst = edge_index[1].astype(jnp.int32)

    npad = E_SORT - E
    dst_pad = jnp.concatenate([dst, jnp.full((npad,), PADV, jnp.int32)])
    eidx = jnp.arange(E_SORT, dtype=jnp.int32)
    dst_s, perm = lax.sort((dst_pad, eidx), num_keys=1)
    src_pad = jnp.concatenate([src, jnp.zeros((npad,), jnp.int32)])
    src_s = jnp.take(src_pad, perm)
    slack = E_PHYS - E_SORT
    dst_phys = jnp.concatenate([dst_s, jnp.full((slack,), PADV, jnp.int32)])
    src_phys = jnp.concatenate([src_s, jnp.zeros((slack,), jnp.int32)])
    perm_phys = jnp.concatenate([perm, jnp.zeros((slack,), jnp.int32)])

    bounds = jnp.searchsorted(
        dst_s, jnp.arange(33, dtype=jnp.int32) * TPN).astype(jnp.int32)
    bounds = jnp.concatenate([bounds, jnp.zeros((15,), jnp.int32)])

    ea_pad = jnp.concatenate(
        [edge_attr, jnp.zeros((npad, edge_attr.shape[1]), jnp.float32)])
    x_pad = jnp.concatenate(
        [x, jnp.zeros((NP - N_NODES, x.shape[1]), jnp.float32)])

    layers = params["layers"]
    we_stack = jnp.stack([l["pre_W"][128:144] for l in layers])
    eprojs = _tc_eproj(ea_pad, we_stack)

    h = _tc_in(x_pad, params["in_W"], params["in_b"])
    deg = None
    for l in range(5):
        lay = layers[l]
        hs, hd = _tc_proj(h, lay["pre_W"][:64], lay["pre_W"][64:128], lay["pre_b"])
        if l == 0:
            ssum, smax, deg = _sc_edge_deg(
                dst_phys, src_phys, perm_phys, hs, hd, eprojs[l], bounds)
            deg = deg[:, None]
        else:
            ssum, smax = _sc_edge(
                dst_phys, src_phys, perm_phys, hs, hd, eprojs[l], bounds)
        pw = lay["post_W"]
        h = _tc_post(h, ssum, smax, deg,
                     pw[:64], pw[64:128], pw[128:192], lay["post_b"])

    return _tc_readout(h, params["ro1_W"], params["ro1_b"],
                       params["ro2_W"], params["ro2_b"])
